# Initial kernel scaffold; baseline (speedup 1.0000x reference)
import jax
import jax.numpy as jnp
from jax.experimental import pallas as pl


def kernel(x, edge_index, edge_weight, W0, W1, W2, b):
    raise NotImplementedError("write your pallas kernel here")


# speedup vs baseline: 299.8441x; 299.8441x over previous
import functools

import jax
import jax.numpy as jnp
from jax import lax
from jax.experimental import pallas as pl
from jax.experimental.pallas import tpu as pltpu
from jax.experimental.pallas import tpu_sc as plsc

N = 100000
E = 6400000
LANES = 128
ROWS = 784
NP = ROWS * LANES
NC = 2
NS = 16
NW = NC * NS
PT = NP // NS
EW = E // NW
CA = 20000
CB = 4000
L = 16

_mesh = plsc.VectorSubcoreMesh(core_axis_name="c", subcore_axis_name="s")
_sc_params = pltpu.CompilerParams(needs_layout_passes=False)


def _zero_shared(zbuf, acc_sh, s):
    def fill(i, carry):
        zbuf[pl.ds(i * L, L)] = jnp.zeros((L,), jnp.float32)
        return carry

    lax.fori_loop(0, PT // L, fill, 0)
    pltpu.sync_copy(zbuf, acc_sh.at[pl.ds(s * PT, PT)])


@functools.partial(
    pl.kernel,
    out_type=jax.ShapeDtypeStruct((NW, PT), jnp.float32),
    mesh=_mesh,
    compiler_params=_sc_params,
    scratch_types=[
        pltpu.VMEM((CA,), jnp.int32),
        pltpu.VMEM((CA,), jnp.float32),
        pltpu.VMEM((PT,), jnp.float32),
        pltpu.VMEM_SHARED((NP,), jnp.float32),
    ],
)
def _degree(col_hbm, w_hbm, out, col_v, w_v, zbuf, acc_sh):
    c = lax.axis_index("c")
    s = lax.axis_index("s")
    wid = c * NS + s

    _zero_shared(zbuf, acc_sh, s)
    plsc.subcore_barrier()

    def chunk(k, carry):
        base = wid * EW + k * CA
        pltpu.sync_copy(col_hbm.at[pl.ds(base, CA)], col_v)
        pltpu.sync_copy(w_hbm.at[pl.ds(base, CA)], w_v)
        pltpu.sync_copy(w_v, acc_sh.at[col_v], add=True)
        return carry

    lax.fori_loop(0, EW // CA, chunk, 0)
    plsc.subcore_barrier()
    pltpu.sync_copy(acc_sh.at[pl.ds(s * PT, PT)], out.at[wid])


@functools.partial(
    pl.kernel,
    out_type=jax.ShapeDtypeStruct((NW, PT), jnp.float32),
    mesh=_mesh,
    compiler_params=_sc_params,
    scratch_types=[
        pltpu.VMEM((NP,), jnp.float32),
        pltpu.VMEM((CB,), jnp.int32),
        pltpu.VMEM((CB,), jnp.int32),
        pltpu.VMEM((CB,), jnp.float32),
        pltpu.VMEM((CB,), jnp.float32),
        pltpu.VMEM((PT,), jnp.float32),
        pltpu.VMEM_SHARED((NP,), jnp.float32),
    ],
)
def _prop(row_hbm, col_hbm, w_hbm, table_hbm, out,
          table_v, row_v, col_v, w_v, contrib_v, zbuf, acc_sh):
    c = lax.axis_index("c")
    s = lax.axis_index("s")
    wid = c * NS + s

    _zero_shared(zbuf, acc_sh, s)
    pltpu.sync_copy(table_hbm, table_v)
    plsc.subcore_barrier()

    def chunk(k, carry):
        base = wid * EW + k * CB
        pltpu.sync_copy(row_hbm.at[pl.ds(base, CB)], row_v)
        pltpu.sync_copy(col_hbm.at[pl.ds(base, CB)], col_v)
        pltpu.sync_copy(w_hbm.at[pl.ds(base, CB)], w_v)

        def grp(j, inner):
            sl = pl.ds(j * L, L)
            g = plsc.load_gather(table_v, [row_v[sl]])
            contrib_v[sl] = g * w_v[sl]
            return inner

        lax.fori_loop(0, CB // L, grp, 0)
        pltpu.sync_copy(contrib_v, acc_sh.at[col_v], add=True)
        return carry

    lax.fori_loop(0, EW // CB, chunk, 0)
    plsc.subcore_barrier()
    pltpu.sync_copy(acc_sh.at[pl.ds(s * PT, PT)], out.at[wid])


def _stage1_body(d0, d1, x0, x1, wv, dinv_o, a0_o, ua1_o, u2_o):
    deg = d0[...] + d1[...]
    pos = deg > 0.0
    safe = jnp.where(pos, deg, 1.0)
    dinv = jnp.where(pos, lax.rsqrt(safe), 0.0)
    dinv_o[...] = dinv
    a0_o[...] = x0[...] * wv[0] + x1[...] * wv[1]
    ua1_o[...] = dinv * (x0[...] * wv[2] + x1[...] * wv[3])
    u2_o[...] = dinv * (x0[...] * wv[4] + x1[...] * wv[5])


def _stage2_body(t0, t1, dinv, ua1, um_o):
    dv = dinv[...]
    um_o[...] = ua1[...] + dv * dv * (t0[...] + t1[...])


def _stage3_body(s0, s1, dinv, a0, wv, out_o):
    out_o[...] = jnp.maximum(
        a0[...] + dinv[...] * (s0[...] + s1[...]) + wv[6], 0.0)


_vspec = pl.BlockSpec(memory_space=pltpu.VMEM)
_sspec = pl.BlockSpec(memory_space=pltpu.SMEM)
_nshape = jax.ShapeDtypeStruct((ROWS, LANES), jnp.float32)

_stage1 = pl.pallas_call(
    _stage1_body,
    out_shape=(_nshape, _nshape, _nshape, _nshape),
    in_specs=[_vspec, _vspec, _vspec, _vspec, _sspec],
    out_specs=(_vspec, _vspec, _vspec, _vspec),
)

_stage2 = pl.pallas_call(
    _stage2_body,
    out_shape=_nshape,
    in_specs=[_vspec, _vspec, _vspec, _vspec],
    out_specs=_vspec,
)

_stage3 = pl.pallas_call(
    _stage3_body,
    out_shape=_nshape,
    in_specs=[_vspec, _vspec, _vspec, _vspec, _sspec],
    out_specs=_vspec,
)


def _halves(parts):
    p = parts.reshape(NC, ROWS, LANES)
    return p[0], p[1]


def kernel(x, edge_index, edge_weight, W0, W1, W2, b):
    row = edge_index[0]
    col = edge_index[1]

    pad = NP - N
    x0 = jnp.pad(x[:, 0], (0, pad)).reshape(ROWS, LANES)
    x1 = jnp.pad(x[:, 1], (0, pad)).reshape(ROWS, LANES)
    wv = jnp.stack([W0[0, 0], W0[1, 0], W1[0, 0], W1[1, 0],
                    W2[0, 0], W2[1, 0], b[0], b[0]])

    deg_parts = _degree(col, edge_weight)
    d0, d1 = _halves(deg_parts)

    dinv, a0, ua1, u2 = _stage1(d0, d1, x0, x1, wv)

    t_parts = _prop(row, col, edge_weight, u2.reshape(NP))
    t0, t1 = _halves(t_parts)
    um = _stage2(t0, t1, dinv, ua1)

    s_parts = _prop(row, col, edge_weight, um.reshape(NP))
    s0, s1 = _halves(s_parts)
    out = _stage3(s0, s1, dinv, a0, wv)

    return out.reshape(NP, 1)[:N]


# speedup vs baseline: 668.4243x; 2.2292x over previous
import functools

import jax
import jax.numpy as jnp
from jax import lax
from jax.experimental import pallas as pl
from jax.experimental.pallas import tpu as pltpu
from jax.experimental.pallas import tpu_sc as plsc

N = 100000
E = 6400000
LANES = 128
ROWS = 784
NP = ROWS * LANES
NC = 2
NS = 16
NW = NC * NS
PT = NP // NS
EW = E // NW
CA = 10000
CB = 1600
ZCH = 1568
L = 16
RING = 4

_mesh = plsc.VectorSubcoreMesh(core_axis_name="c", subcore_axis_name="s")
_sc_params = pltpu.CompilerParams(needs_layout_passes=False)


def _zero_shared(buf, acc_sh, s):
    def fill(i, carry):
        buf[pl.ds(i * L, L)] = jnp.zeros((L,), jnp.float32)
        return carry

    lax.fori_loop(0, ZCH // L, fill, 0)
    for q in range(4):
        pltpu.sync_copy(buf.at[pl.ds(0, ZCH)],
                        acc_sh.at[pl.ds(s * PT + q * ZCH, ZCH)])


@functools.partial(
    pl.kernel,
    out_type=jax.ShapeDtypeStruct((NW, PT), jnp.float32),
    mesh=_mesh,
    compiler_params=_sc_params,
    scratch_types=[
        [pltpu.VMEM((CA,), jnp.int32) for _ in range(RING)],
        [pltpu.VMEM((CA,), jnp.float32) for _ in range(RING)],
        pltpu.VMEM_SHARED((NP,), jnp.float32),
        [pltpu.SemaphoreType.DMA for _ in range(RING)],
        [pltpu.SemaphoreType.DMA for _ in range(RING)],
    ],
)
def _degree(col_hbm, w_hbm, out, col_b, w_b, acc_sh, in_sems, sc_sems):
    c = lax.axis_index("c")
    s = lax.axis_index("s")
    wid = c * NS + s
    nch = EW // CA

    def in_start(k, r):
        base = wid * EW + k * CA
        pltpu.async_copy(col_hbm.at[pl.ds(base, CA)], col_b[r], in_sems[r])
        pltpu.async_copy(w_hbm.at[pl.ds(base, CA)], w_b[r], in_sems[r])

    def in_wait(r):
        pltpu.make_async_copy(col_hbm.at[pl.ds(0, CA)], col_b[r], in_sems[r]).wait()
        pltpu.make_async_copy(w_hbm.at[pl.ds(0, CA)], w_b[r], in_sems[r]).wait()

    def sc_wait(r):
        pltpu.make_async_copy(w_b[r], acc_sh.at[col_b[r]], sc_sems[r]).wait()

    in_start(0, 0)
    in_start(1, 1)
    _zero_shared(w_b[3], acc_sh, s)
    plsc.subcore_barrier()

    def step(t, carry):
        for u in range(RING):
            k = t * RING + u
            in_wait(u)
            pltpu.async_copy(w_b[u], acc_sh.at[col_b[u]], sc_sems[u], add=True)
            nxt = (u + 2) % RING

            @pl.when(k >= 2)
            def _():
                sc_wait(nxt)

            @pl.when(k + 2 < nch)
            def _():
                in_start(k + 2, nxt)

        return carry

    lax.fori_loop(0, nch // RING, step, 0)
    for k in range(nch - 2, nch):
        sc_wait(k % RING)
    plsc.subcore_barrier()
    pltpu.sync_copy(acc_sh.at[pl.ds(s * PT, PT)], out.at[wid])


@functools.partial(
    pl.kernel,
    out_type=jax.ShapeDtypeStruct((NW, PT), jnp.float32),
    mesh=_mesh,
    compiler_params=_sc_params,
    scratch_types=[
        pltpu.VMEM((NP,), jnp.float32),
        [pltpu.VMEM((CB,), jnp.int32) for _ in range(RING)],
        [pltpu.VMEM((CB,), jnp.int32) for _ in range(RING)],
        [pltpu.VMEM((CB,), jnp.float32) for _ in range(RING)],
        pltpu.VMEM_SHARED((NP,), jnp.float32),
        [pltpu.SemaphoreType.DMA for _ in range(RING)],
        [pltpu.SemaphoreType.DMA for _ in range(RING)],
    ],
)
def _prop(row_hbm, col_hbm, w_hbm, table_hbm, out,
          table_v, row_b, col_b, w_b, acc_sh, in_sems, sc_sems):
    c = lax.axis_index("c")
    s = lax.axis_index("s")
    wid = c * NS + s
    nch = EW // CB

    def in_start(k, r):
        base = wid * EW + k * CB
        pltpu.async_copy(row_hbm.at[pl.ds(base, CB)], row_b[r], in_sems[r])
        pltpu.async_copy(col_hbm.at[pl.ds(base, CB)], col_b[r], in_sems[r])
        pltpu.async_copy(w_hbm.at[pl.ds(base, CB)], w_b[r], in_sems[r])

    def in_wait(r):
        pltpu.make_async_copy(row_hbm.at[pl.ds(0, CB)], row_b[r], in_sems[r]).wait()
        pltpu.make_async_copy(col_hbm.at[pl.ds(0, CB)], col_b[r], in_sems[r]).wait()
        pltpu.make_async_copy(w_hbm.at[pl.ds(0, CB)], w_b[r], in_sems[r]).wait()

    def sc_wait(r):
        pltpu.make_async_copy(w_b[r], acc_sh.at[col_b[r]], sc_sems[r]).wait()

    in_start(0, 0)
    in_start(1, 1)
    pltpu.sync_copy(table_hbm, table_v)
    _zero_shared(w_b[3], acc_sh, s)
    plsc.subcore_barrier()

    def do_chunk(k, u):
        in_wait(u)

        @plsc.parallel_loop(0, CB // L, unroll=5)
        def grp(j):
            sl = pl.ds(j * L, L)
            g = plsc.load_gather(table_v, [row_b[u][sl]])
            w_b[u][sl] = g * w_b[u][sl]

        pltpu.async_copy(w_b[u], acc_sh.at[col_b[u]], sc_sems[u], add=True)
        nxt = (u + 2) % RING

        if isinstance(k, int):
            if k >= 2:
                sc_wait(nxt)
            if k + 2 < nch:
                in_start(k + 2, nxt)
        else:
            @pl.when(k >= 2)
            def _():
                sc_wait(nxt)

            @pl.when(k + 2 < nch)
            def _():
                in_start(k + 2, nxt)

    def step(t, carry):
        for u in range(RING):
            do_chunk(t * RING + u, u)
        return carry

    full = (nch // RING) * RING
    lax.fori_loop(0, nch // RING, step, 0)
    for k in range(full, nch):
        do_chunk(k, k % RING)
    for k in range(nch - 2, nch):
        sc_wait(k % RING)
    plsc.subcore_barrier()
    pltpu.sync_copy(acc_sh.at[pl.ds(s * PT, PT)], out.at[wid])


def _stage1_body(d0, d1, x0, x1, wv, dinv_o, a0_o, ua1_o, u2_o):
    deg = d0[...] + d1[...]
    pos = deg > 0.0
    safe = jnp.where(pos, deg, 1.0)
    dinv = jnp.where(pos, lax.rsqrt(safe), 0.0)
    dinv_o[...] = dinv
    a0_o[...] = x0[...] * wv[0] + x1[...] * wv[1]
    ua1_o[...] = dinv * (x0[...] * wv[2] + x1[...] * wv[3])
    u2_o[...] = dinv * (x0[...] * wv[4] + x1[...] * wv[5])


def _stage2_body(t0, t1, dinv, ua1, um_o):
    dv = dinv[...]
    um_o[...] = ua1[...] + dv * dv * (t0[...] + t1[...])


def _stage3_body(s0, s1, dinv, a0, wv, out_o):
    out_o[...] = jnp.maximum(
        a0[...] + dinv[...] * (s0[...] + s1[...]) + wv[6], 0.0)


_vspec = pl.BlockSpec(memory_space=pltpu.VMEM)
_sspec = pl.BlockSpec(memory_space=pltpu.SMEM)
_nshape = jax.ShapeDtypeStruct((ROWS, LANES), jnp.float32)

_stage1 = pl.pallas_call(
    _stage1_body,
    out_shape=(_nshape, _nshape, _nshape, _nshape),
    in_specs=[_vspec, _vspec, _vspec, _vspec, _sspec],
    out_specs=(_vspec, _vspec, _vspec, _vspec),
)

_stage2 = pl.pallas_call(
    _stage2_body,
    out_shape=_nshape,
    in_specs=[_vspec, _vspec, _vspec, _vspec],
    out_specs=_vspec,
)

_stage3 = pl.pallas_call(
    _stage3_body,
    out_shape=_nshape,
    in_specs=[_vspec, _vspec, _vspec, _vspec, _sspec],
    out_specs=_vspec,
)


def _halves(parts):
    p = parts.reshape(NC, ROWS, LANES)
    return p[0], p[1]


def kernel(x, edge_index, edge_weight, W0, W1, W2, b):
    row = edge_index[0]
    col = edge_index[1]

    pad = NP - N
    x0 = jnp.pad(x[:, 0], (0, pad)).reshape(ROWS, LANES)
    x1 = jnp.pad(x[:, 1], (0, pad)).reshape(ROWS, LANES)
    wv = jnp.stack([W0[0, 0], W0[1, 0], W1[0, 0], W1[1, 0],
                    W2[0, 0], W2[1, 0], b[0], b[0]])

    deg_parts = _degree(col, edge_weight)
    d0, d1 = _halves(deg_parts)

    dinv, a0, ua1, u2 = _stage1(d0, d1, x0, x1, wv)

    t_parts = _prop(row, col, edge_weight, u2.reshape(NP))
    t0, t1 = _halves(t_parts)
    um = _stage2(t0, t1, dinv, ua1)

    s_parts = _prop(row, col, edge_weight, um.reshape(NP))
    s0, s1 = _halves(s_parts)
    out = _stage3(s0, s1, dinv, a0, wv)

    return out.reshape(NP, 1)[:N]
